# Initial kernel scaffold; baseline (speedup 1.0000x reference)
import jax
import jax.numpy as jnp
from jax.experimental import pallas as pl


def kernel(x, edge_index, W, b):
    raise NotImplementedError("write your pallas kernel here")


# speedup vs baseline: 3.4526x; 3.4526x over previous
import functools

import jax
import jax.numpy as jnp
from jax import lax
from jax.experimental import pallas as pl
from jax.experimental.pallas import tpu as pltpu
from jax.experimental.pallas import tpu_sc as plsc

_N = 10000
_E = 160000
_D = 256
_DH = 128

_NSUB = 16
_CHUNK = 128
_EPT = 10112
_NCHUNK = _EPT // _CHUNK
_EPAD = _EPT * _NSUB
_AGG_ROWS = 10112
_ZROWS = _AGG_ROWS // _NSUB
_OROWS = 624


def _sc_aggregate(xh, srcs, dstp, zrows):
  mesh = plsc.VectorSubcoreMesh(core_axis_name="c", subcore_axis_name="s")

  @functools.partial(
      pl.kernel,
      out_type=jax.ShapeDtypeStruct((2, _N, _DH), jnp.float32),
      mesh=mesh,
      scratch_types=[
          pltpu.VMEM((_CHUNK,), jnp.int32),
          pltpu.VMEM((_CHUNK,), jnp.int32),
          pltpu.VMEM((_CHUNK, _DH), jnp.float32),
          pltpu.VMEM_SHARED((_AGG_ROWS, _DH), jnp.float32),
          pltpu.SemaphoreType.DMA,
      ],
  )
  def k(xh_hbm, srcs_hbm, dst_hbm, z_hbm, out_hbm, src_v, dst_v, rows_v,
        agg, sem):
    c = lax.axis_index("c")
    s = lax.axis_index("s")

    pltpu.sync_copy(z_hbm, agg.at[pl.ds(s * _ZROWS, _ZROWS)])
    plsc.subcore_barrier()

    def body(j, _):
      off = s * _EPT + j * _CHUNK
      pltpu.sync_copy(srcs_hbm.at[pl.ds(c * _EPAD + off, _CHUNK)], src_v)
      pltpu.sync_copy(dst_hbm.at[pl.ds(off, _CHUNK)], dst_v)
      pltpu.async_copy(xh_hbm.at[src_v], rows_v, sem).wait()
      pltpu.sync_copy(rows_v, agg.at[dst_v], add=True)
      return ()

    lax.fori_loop(0, _NCHUNK, body, ())
    plsc.subcore_barrier()

    pltpu.sync_copy(agg.at[pl.ds(s * _OROWS, _OROWS)],
                    out_hbm.at[c, pl.ds(s * _OROWS, _OROWS)])

    @pl.when(s == _NSUB - 1)
    def _epilogue():
      tail = _NSUB * _OROWS
      pltpu.sync_copy(agg.at[pl.ds(tail, _N - tail)],
                      out_hbm.at[c, pl.ds(tail, _N - tail)])

  return k(xh, srcs, dstp, zrows)


def _tc_linear_body(a0_ref, a1_ref, w0_ref, w1_ref, b_ref, o_ref):
  dn = (((1,), (1,)), ((), ()))
  acc = lax.dot_general(a0_ref[0], w0_ref[...], dn,
                        preferred_element_type=jnp.float32)
  acc += lax.dot_general(a1_ref[0], w1_ref[...], dn,
                         preferred_element_type=jnp.float32)
  o_ref[...] = jnp.maximum(acc + b_ref[...], 0.0)


def _tc_linear(agg2, w0, w1, b2):
  rows = 1000
  grid = _N // rows
  return pl.pallas_call(
      _tc_linear_body,
      grid=(grid,),
      in_specs=[
          pl.BlockSpec((1, rows, _DH), lambda i: (0, i, 0)),
          pl.BlockSpec((1, rows, _DH), lambda i: (1, i, 0)),
          pl.BlockSpec((_D, _DH), lambda i: (0, 0)),
          pl.BlockSpec((_D, _DH), lambda i: (0, 0)),
          pl.BlockSpec((1, _D), lambda i: (0, 0)),
      ],
      out_specs=pl.BlockSpec((rows, _D), lambda i: (i, 0)),
      out_shape=jax.ShapeDtypeStruct((_N, _D), jnp.float32),
  )(agg2, agg2, w0, w1, b2)


def kernel(x, edge_index, W, b):
  src = edge_index[0].astype(jnp.int32)
  dst = edge_index[1].astype(jnp.int32)
  pad = _EPAD - _E
  srcp = jnp.concatenate([src, jnp.zeros((pad,), jnp.int32)])
  dstp = jnp.concatenate([dst, jnp.full((pad,), _N, jnp.int32)])
  srcs = jnp.concatenate([srcp, srcp + _N])
  xh = jnp.concatenate([x[:, :_DH], x[:, _DH:]], axis=0)
  zrows = jnp.zeros((_ZROWS, _DH), jnp.float32)

  agg2 = _sc_aggregate(xh, srcs, dstp, zrows)

  w0 = W[:, :_DH]
  w1 = W[:, _DH:]
  b2 = b.reshape(1, _D)
  return _tc_linear(agg2, w0, w1, b2)


# speedup vs baseline: 3.6023x; 1.0433x over previous
import functools

import jax
import jax.numpy as jnp
from jax import lax
from jax.experimental import pallas as pl
from jax.experimental.pallas import tpu as pltpu
from jax.experimental.pallas import tpu_sc as plsc

_N = 10000
_E = 160000
_D = 256
_DH = 128

_NSUB = 16
_CHUNK = 128
_NCHUNK = 80
_EPT = _NCHUNK * _CHUNK
_EPAD = _EPT * _NSUB
_NBUF = 2
_AGG_ROWS = 10112
_ZROWS = _AGG_ROWS // _NSUB
_OROWS = 624


def _sc_aggregate(xh, srcs, dst3, zrows):
  mesh = plsc.VectorSubcoreMesh(core_axis_name="c", subcore_axis_name="s")

  @functools.partial(
      pl.kernel,
      out_type=jax.ShapeDtypeStruct((2, _N, _DH), jnp.float32),
      mesh=mesh,
      scratch_types=[
          pltpu.VMEM((_NCHUNK, _CHUNK), jnp.int32),
          pltpu.VMEM((_CHUNK,), jnp.int32),
          [pltpu.VMEM((_CHUNK, _DH), jnp.float32) for _ in range(_NBUF)],
          [pltpu.SemaphoreType.DMA for _ in range(_NBUF)],
          pltpu.VMEM_SHARED((_AGG_ROWS, _DH), jnp.float32),
      ],
  )
  def k(xh_hbm, srcs_hbm, dst_hbm, z_hbm, out_hbm, src_all, dst_v,
        bufs, sems, agg):
    c = lax.axis_index("c")
    s = lax.axis_index("s")

    pltpu.sync_copy(z_hbm, agg.at[pl.ds(s * _ZROWS, _ZROWS)])
    pltpu.sync_copy(srcs_hbm.at[c, s], src_all)
    plsc.subcore_barrier()

    def start(jj, b):
      pltpu.async_copy(xh_hbm.at[src_all.at[jj]], bufs[b], sems[b])

    def wait(b):
      pltpu.make_async_copy(xh_hbm.at[src_all.at[0]], bufs[b],
                            sems[b]).wait()

    for b in range(_NBUF):
      start(b, b)

    iters = (_NCHUNK - _NBUF) // _NBUF

    def body(i, _):
      base = i * _NBUF
      for b in range(_NBUF):
        jj = base + b
        wait(b)
        pltpu.sync_copy(dst_hbm.at[s, jj], dst_v)
        pltpu.sync_copy(bufs[b], agg.at[dst_v], add=True)
        start(jj + _NBUF, b)
      return ()

    lax.fori_loop(0, iters, body, ())
    for b in range(_NBUF):
      jj = _NCHUNK - _NBUF + b
      wait(b)
      pltpu.sync_copy(dst_hbm.at[s, jj], dst_v)
      pltpu.sync_copy(bufs[b], agg.at[dst_v], add=True)

    plsc.subcore_barrier()

    pltpu.sync_copy(agg.at[pl.ds(s * _OROWS, _OROWS)],
                    out_hbm.at[c, pl.ds(s * _OROWS, _OROWS)])

    @pl.when(s == _NSUB - 1)
    def _epilogue():
      tail = _NSUB * _OROWS
      pltpu.sync_copy(agg.at[pl.ds(tail, _N - tail)],
                      out_hbm.at[c, pl.ds(tail, _N - tail)])

  return k(xh, srcs, dst3, zrows)


def _tc_linear_body(a0_ref, a1_ref, w0_ref, w1_ref, b_ref, o_ref):
  dn = (((1,), (1,)), ((), ()))
  acc = lax.dot_general(a0_ref[0], w0_ref[...], dn,
                        preferred_element_type=jnp.float32)
  acc += lax.dot_general(a1_ref[0], w1_ref[...], dn,
                         preferred_element_type=jnp.float32)
  o_ref[...] = jnp.maximum(acc + b_ref[...], 0.0)


def _tc_linear(agg2, w0, w1, b2):
  rows = 1000
  grid = _N // rows
  return pl.pallas_call(
      _tc_linear_body,
      grid=(grid,),
      in_specs=[
          pl.BlockSpec((1, rows, _DH), lambda i: (0, i, 0)),
          pl.BlockSpec((1, rows, _DH), lambda i: (1, i, 0)),
          pl.BlockSpec((_D, _DH), lambda i: (0, 0)),
          pl.BlockSpec((_D, _DH), lambda i: (0, 0)),
          pl.BlockSpec((1, _D), lambda i: (0, 0)),
      ],
      out_specs=pl.BlockSpec((rows, _D), lambda i: (i, 0)),
      out_shape=jax.ShapeDtypeStruct((_N, _D), jnp.float32),
  )(agg2, agg2, w0, w1, b2)


def kernel(x, edge_index, W, b):
  src = edge_index[0].astype(jnp.int32)
  dst = edge_index[1].astype(jnp.int32)
  pad = _EPAD - _E
  srcp = jnp.concatenate([src, jnp.zeros((pad,), jnp.int32)])
  dstp = jnp.concatenate([dst, jnp.full((pad,), _N, jnp.int32)])
  srcs = jnp.concatenate([srcp, srcp + _N]).reshape(
      2, _NSUB, _NCHUNK, _CHUNK)
  dst3 = dstp.reshape(_NSUB, _NCHUNK, _CHUNK)
  xh = jnp.concatenate([x[:, :_DH], x[:, _DH:]], axis=0)
  zrows = jnp.zeros((_ZROWS, _DH), jnp.float32)

  agg2 = _sc_aggregate(xh, srcs, dst3, zrows)

  w0 = W[:, :_DH]
  w1 = W[:, _DH:]
  b2 = b.reshape(1, _D)
  return _tc_linear(agg2, w0, w1, b2)
